# Initial kernel scaffold; baseline (speedup 1.0000x reference)
import jax
import jax.numpy as jnp
from jax.experimental import pallas as pl


def kernel(sp, cx, wc, bc, bns, bnb, w1, b1, w2, b2):
    raise NotImplementedError("write your pallas kernel here")


# speedup vs baseline: 1.3164x; 1.3164x over previous
import functools

import jax
import jax.numpy as jnp
from jax import lax
from jax.experimental import pallas as pl
from jax.experimental.pallas import tpu as pltpu


def _round_up(x, m):
    return ((x + m - 1) // m) * m


def _ffm_kernel(sp_ref, cx_ref, wk_ref, bcb_ref, w1_ref, b1_ref, w2_ref,
                b2_ref, out_ref, xpad_ref, *, H, W, B):
    HW = H * W
    Cout = out_ref.shape[1]
    base = W + 1

    col = lax.broadcasted_iota(jnp.int32, (1, HW), 1) % W
    left_ok = col != 0
    right_ok = col != W - 1

    for b in range(B):
        if b == 0:
            xpad_ref[...] = jnp.zeros(xpad_ref.shape, xpad_ref.dtype)
        x = (sp_ref[b] + cx_ref[b]).astype(jnp.bfloat16)
        xpad_ref[:, base:base + HW] = x

        slices = []
        for kh in range(3):
            for kw in range(3):
                o = kh * W + kw
                s = xpad_ref[:, o:o + HW]
                if kw == 0:
                    s = jnp.where(left_ok, s, 0)
                elif kw == 2:
                    s = jnp.where(right_ok, s, 0)
                slices.append(s)
        rhs = jnp.concatenate(slices, axis=0)

        acc = jnp.dot(wk_ref[...], rhs, preferred_element_type=jnp.float32)
        feather = jnp.maximum(acc + bcb_ref[...], 0.0)

        pooled = jnp.sum(feather, axis=1, keepdims=True) * (1.0 / HW)
        pooled_b = jnp.broadcast_to(pooled, (Cout, 128))
        h1 = jnp.maximum(
            jnp.dot(w1_ref[...], pooled_b, preferred_element_type=jnp.float32)
            + b1_ref[...], 0.0)
        z = (jnp.dot(w2_ref[...], h1, preferred_element_type=jnp.float32)
             + b2_ref[...])
        gate = 1.0 + jax.nn.sigmoid(z[:, 0:1])

        out_ref[b] = feather * gate


@jax.jit
def _ffm(sp, cx, wc, bc, bns, bnb, w1, b1, w2, b2):
    N, Cin, H, W = sp.shape
    Cout = w1.shape[0]
    HW = H * W
    Lpad = _round_up(HW + 2 * W + 2, 128)
    B = 4
    assert N % B == 0

    bns_c = bns.reshape(Cout, 1)
    bnb_c = bnb.reshape(Cout, 1)
    wk = jnp.transpose(wc.reshape(9, Cin, Cout), (0, 2, 1)) * (0.5 * bns_c)[None]
    wk_flat = jnp.transpose(wk, (1, 0, 2)).reshape(Cout, 9 * Cin)
    wk_bf = wk_flat.astype(jnp.bfloat16)
    bcb = bc.reshape(Cout, 1) * bns_c + bnb_c

    w1c = w1.T
    b1c = b1.reshape(Cout, 1)
    w2c = w2.T
    b2c = b2.reshape(Cout, 1)

    spf = sp.reshape(N, Cin, HW)
    cxf = cx.reshape(N, Cin, HW)

    kernel_fn = functools.partial(_ffm_kernel, H=H, W=W, B=B)
    out = pl.pallas_call(
        kernel_fn,
        out_shape=jax.ShapeDtypeStruct((N, Cout, HW), jnp.float32),
        grid=(N // B,),
        in_specs=[
            pl.BlockSpec((B, Cin, HW), lambda i: (i, 0, 0)),
            pl.BlockSpec((B, Cin, HW), lambda i: (i, 0, 0)),
            pl.BlockSpec((Cout, 9 * Cin), lambda i: (0, 0)),
            pl.BlockSpec((Cout, 1), lambda i: (0, 0)),
            pl.BlockSpec((Cout, Cout), lambda i: (0, 0)),
            pl.BlockSpec((Cout, 1), lambda i: (0, 0)),
            pl.BlockSpec((Cout, Cout), lambda i: (0, 0)),
            pl.BlockSpec((Cout, 1), lambda i: (0, 0)),
        ],
        out_specs=pl.BlockSpec((B, Cout, HW), lambda i: (i, 0, 0)),
        scratch_shapes=[pltpu.VMEM((Cin, Lpad), jnp.bfloat16)],
        compiler_params=pltpu.CompilerParams(
            dimension_semantics=("parallel",)),
        cost_estimate=pl.CostEstimate(
            flops=2 * N * 9 * Cout * Cin * HW + 2 * N * 2 * Cout * Cout * 128,
            transcendentals=N * Cout,
            bytes_accessed=4 * (2 * N * Cin * HW + N * Cout * HW
                                + 2 * Cout * Cout + 3 * Cout)
                           + 2 * Cout * 9 * Cin),
    )(spf, cxf, wk_bf, bcb, w1c, b1c, w2c, b2c)

    return out.reshape(N, Cout, H, W)


def kernel(sp, cx, wc, bc, bns, bnb, w1, b1, w2, b2):
    return _ffm(sp, cx, wc, bc, bns, bnb, w1, b1, w2, b2)


# speedup vs baseline: 1.4523x; 1.1032x over previous
import functools

import jax
import jax.numpy as jnp
from jax import lax
from jax.experimental import pallas as pl
from jax.experimental.pallas import tpu as pltpu


def _round_up(x, m):
    return ((x + m - 1) // m) * m


def _ffm_kernel(xpf_ref, wk_ref, bcb_ref, w1_ref, b1_ref, w2_ref,
                b2_ref, out_ref, *, H, W, B):
    HW = H * W
    Cout = out_ref.shape[1]

    col = lax.broadcasted_iota(jnp.int32, (1, HW), 1) % W
    left_ok = col != 0
    right_ok = col != W - 1

    for b in range(B):
        slices = []
        for kh in range(3):
            for kw in range(3):
                o = kh * W + kw
                s = xpf_ref[b, :, o:o + HW]
                if kw == 0:
                    s = jnp.where(left_ok, s, 0)
                elif kw == 2:
                    s = jnp.where(right_ok, s, 0)
                slices.append(s)
        rhs = jnp.concatenate(slices, axis=0)

        acc = lax.dot_general(
            wk_ref[...], rhs,
            dimension_numbers=(((0,), (0,)), ((), ())),
            preferred_element_type=jnp.float32)
        feather = jnp.maximum(acc + bcb_ref[...], 0.0)

        pooled = jnp.sum(feather, axis=1, keepdims=True) * (1.0 / HW)
        pooled_b = jnp.broadcast_to(pooled, (Cout, 128))
        h1 = jnp.maximum(
            jnp.dot(w1_ref[...], pooled_b, preferred_element_type=jnp.float32)
            + b1_ref[...], 0.0)
        z = jnp.dot(w2_ref[...], h1, preferred_element_type=jnp.float32)
        z0 = z[:, 0:1] + b2_ref[...]
        gate = 1.0 + jax.nn.sigmoid(z0)

        out_ref[b] = (feather * gate).astype(jnp.bfloat16)


@jax.jit
def _ffm(sp, cx, wc, bc, bns, bnb, w1, b1, w2, b2):
    N, Cin, H, W = sp.shape
    Cout = w1.shape[0]
    HW = H * W
    base = W + 1
    Lpad = _round_up(HW + 2 * W + 2, 128)
    B = next(b for b in (8, 4, 2, 1) if N % b == 0)

    bns_r = bns.reshape(1, 1, Cout)
    wk = (wc.reshape(9 * Cin, Cout) * (0.5 * bns.reshape(1, Cout))
          ).astype(jnp.bfloat16)
    bcb = bc.reshape(Cout, 1) * bns.reshape(Cout, 1) + bnb.reshape(Cout, 1)
    del bns_r

    w1c = w1.T
    b1c = b1.reshape(Cout, 1)
    w2c = w2.T
    b2c = b2.reshape(Cout, 1)

    x = (sp + cx).reshape(N, Cin, HW).astype(jnp.bfloat16)
    xpf = jnp.zeros((N, Cin, Lpad), jnp.bfloat16)
    xpf = xpf.at[:, :, base:base + HW].set(x)

    kernel_fn = functools.partial(_ffm_kernel, H=H, W=W, B=B)
    out = pl.pallas_call(
        kernel_fn,
        out_shape=jax.ShapeDtypeStruct((N, Cout, HW), jnp.bfloat16),
        grid=(N // B,),
        in_specs=[
            pl.BlockSpec((B, Cin, Lpad), lambda i: (i, 0, 0)),
            pl.BlockSpec((9 * Cin, Cout), lambda i: (0, 0)),
            pl.BlockSpec((Cout, 1), lambda i: (0, 0)),
            pl.BlockSpec((Cout, Cout), lambda i: (0, 0)),
            pl.BlockSpec((Cout, 1), lambda i: (0, 0)),
            pl.BlockSpec((Cout, Cout), lambda i: (0, 0)),
            pl.BlockSpec((Cout, 1), lambda i: (0, 0)),
        ],
        out_specs=pl.BlockSpec((B, Cout, HW), lambda i: (i, 0, 0)),
        compiler_params=pltpu.CompilerParams(
            dimension_semantics=("parallel",)),
        cost_estimate=pl.CostEstimate(
            flops=2 * N * 9 * Cout * Cin * HW + 2 * N * 2 * Cout * Cout * 128,
            transcendentals=N * Cout,
            bytes_accessed=2 * (N * Cin * Lpad + N * Cout * HW)
                           + 4 * (2 * Cout * Cout + 3 * Cout)
                           + 2 * Cout * 9 * Cin),
    )(xpf, wk, bcb, w1c, b1c, w2c, b2c)

    return out.reshape(N, Cout, H, W).astype(jnp.float32)


def kernel(sp, cx, wc, bc, bns, bnb, w1, b1, w2, b2):
    return _ffm(sp, cx, wc, bc, bns, bnb, w1, b1, w2, b2)
